# Initial kernel scaffold; baseline (speedup 1.0000x reference)
import jax
import jax.numpy as jnp
from jax.experimental import pallas as pl


def kernel(x, table, W1, b1, W2, b2):
    raise NotImplementedError("write your pallas kernel here")


# speedup vs baseline: 2.1179x; 2.1179x over previous
import functools

import jax
import jax.numpy as jnp
from jax import lax
from jax.experimental import pallas as pl
from jax.experimental.pallas import tpu as pltpu
from jax.experimental.pallas import tpu_sc as plsc


def _sc_pool_sum(x3d, table, num_cores, num_subcores, lanes):
    NW, BPW, H = x3d.shape
    V, D = table.shape
    DV = D // lanes

    C = 2
    CH = C * H
    NCHUNK = BPW // C
    x4d = x3d.reshape(NW, NCHUNK, CH)

    mesh = plsc.VectorSubcoreMesh(core_axis_name="c", subcore_axis_name="s")

    @functools.partial(
        pl.kernel,
        out_type=jax.ShapeDtypeStruct((NW * BPW, D), jnp.float32),
        mesh=mesh,
        scratch_types=[
            pltpu.VMEM((NCHUNK, CH), jnp.int32),
            pltpu.VMEM((CH, D), jnp.float32),
            pltpu.VMEM((BPW, D), jnp.float32),
            pltpu.SemaphoreType.DMA,
            pltpu.SemaphoreType.DMA,
        ],
        compiler_params=pltpu.CompilerParams(use_tc_tiling_on_sc=False),
    )
    def k(x_hbm, tab_hbm, out_hbm, idx_v, rows_v, pool_v, sem_g, sem_o):
        wid = lax.axis_index("s") * num_cores + lax.axis_index("c")
        pltpu.sync_copy(x_hbm.at[wid], idx_v)

        def chunk_body(g):
            pltpu.async_copy(tab_hbm.at[idx_v.at[g]], rows_v, sem_g).wait()
            for c in range(C):
                def acc_body(r, accs):
                    return tuple(
                        accs[d] + rows_v[c * H + r, pl.ds(d * lanes, lanes)]
                        for d in range(DV)
                    )
                init = tuple(
                    rows_v[c * H, pl.ds(d * lanes, lanes)]
                    for d in range(DV)
                )
                accs = lax.fori_loop(1, H, acc_body, init)
                for d in range(DV):
                    pool_v[g * C + c, pl.ds(d * lanes, lanes)] = accs[d]

        lax.fori_loop(0, NCHUNK, lambda g, _: (chunk_body(g), None)[1], None)

        base = wid * BPW
        pltpu.async_copy(pool_v, out_hbm.at[pl.ds(base, BPW), :], sem_o).wait()

    return k(x4d, table)


def _tc_mlp(pooled_sum, W1, b1, W2, b2, inv_h):
    B, D = pooled_sum.shape
    F = W1.shape[0]
    NO = W2.shape[0]
    bm = 2048

    def body(p_ref, w1_ref, b1_ref, w2_ref, b2_ref, o_ref):
        p = p_ref[...] * inv_h
        h = lax.dot_general(
            p, w1_ref[...], (((1,), (1,)), ((), ())),
            preferred_element_type=jnp.float32,
        ) + b1_ref[...]
        h = jnp.maximum(h, 0.0)
        o = lax.dot_general(
            h, w2_ref[...], (((1,), (1,)), ((), ())),
            preferred_element_type=jnp.float32,
        ) + b2_ref[0, 0]
        o_ref[...] = jax.nn.sigmoid(o)

    return pl.pallas_call(
        body,
        grid=(B // bm,),
        in_specs=[
            pl.BlockSpec((bm, D), lambda i: (i, 0)),
            pl.BlockSpec((F, D), lambda i: (0, 0)),
            pl.BlockSpec((1, F), lambda i: (0, 0)),
            pl.BlockSpec((NO, F), lambda i: (0, 0)),
            pl.BlockSpec(memory_space=pltpu.SMEM),
        ],
        out_specs=pl.BlockSpec((bm, NO), lambda i: (i, 0)),
        out_shape=jax.ShapeDtypeStruct((B, NO), jnp.float32),
    )(pooled_sum, W1, b1.reshape(1, F), W2, b2.reshape(1, 1))


def kernel(x, table, W1, b1, W2, b2):
    B, H = x.shape
    V, D = table.shape
    info = plsc.get_sparse_core_info()
    NW = info.num_cores * info.num_subcores
    BPW = B // NW
    x3d = x.reshape(NW, BPW, H)
    pooled_sum = _sc_pool_sum(x3d, table, info.num_cores, info.num_subcores,
                              info.num_lanes)
    W2p = jnp.pad(W2, ((0, 8 - W2.shape[0]), (0, 0)))
    out = _tc_mlp(pooled_sum, W1, b1, W2p, b2, 1.0 / H)
    return out[:, : W2.shape[0]]
